# baseline (speedup 1.0000x reference)
import functools
import math

import jax
import jax.numpy as jnp
from jax.experimental import pallas as pl
from jax.experimental.pallas import tpu as pltpu


def _router_body(capacity, x_ref, cw_ref, mask_ref,
                 flat1_ref, w1_ref, flat2_ref, w2_ref):
    i = pl.program_id(0)
    T, E = x_ref.shape
    TB = cw_ref.shape[0]

    @pl.when(i == 0)
    def _compute_metadata():
        x = x_ref[...]
        m = jnp.max(x, axis=1, keepdims=True)
        ex = jnp.exp(x - m)
        p = ex / jnp.sum(ex, axis=1, keepdims=True)

        idx1 = jnp.argmax(p, axis=1).astype(jnp.int32)[:, None]
        p1 = jnp.max(p, axis=1, keepdims=True)
        eids = jax.lax.broadcasted_iota(jnp.int32, (T, E), 1)
        m1 = eids == idx1
        pm = jnp.where(m1, -jnp.inf, p)
        idx2 = jnp.argmax(pm, axis=1).astype(jnp.int32)[:, None]
        p2 = jnp.max(pm, axis=1, keepdims=True)
        m2 = eids == idx2

        def _cumsum0(v):
            s = 1
            while s < v.shape[0]:
                z = jnp.zeros((s, v.shape[1]), v.dtype)
                v = v + jnp.concatenate([z, v[:-s]], axis=0)
                s *= 2
            return v

        c1 = _cumsum0(m1.astype(jnp.int32))
        c2 = _cumsum0(m2.astype(jnp.int32))
        total1 = c1[T - 1:T, :]
        rank1 = jnp.sum(jnp.where(m1, c1, 0), axis=1, keepdims=True) - 1
        rank2 = jnp.sum(jnp.where(m2, c2 + total1, 0), axis=1, keepdims=True) - 1

        flat1 = jnp.where(rank1 < capacity, idx1 * capacity + rank1, -1)
        flat2 = jnp.where(rank2 < capacity, idx2 * capacity + rank2, -1)
        flat1_ref[...] = flat1
        flat2_ref[...] = flat2
        w1_ref[...] = p1
        w2_ref[...] = p2

    f1 = flat1_ref[pl.ds(i * TB, TB), :][:, :, None]
    f2 = flat2_ref[pl.ds(i * TB, TB), :][:, :, None]
    w1 = w1_ref[pl.ds(i * TB, TB), :][:, :, None]
    w2 = w2_ref[pl.ds(i * TB, TB), :][:, :, None]
    je = jax.lax.broadcasted_iota(jnp.int32, (TB, E, capacity), 1)
    jc = jax.lax.broadcasted_iota(jnp.int32, (TB, E, capacity), 2)
    J = je * capacity + jc
    out = jnp.where(J == f1, w1, 0.0)
    out = jnp.where(J == f2, w2, out)
    cw_ref[...] = out
    mask_ref[...] = out != 0.0


@jax.jit
def kernel(inputs):
    T, E = inputs.shape
    capacity = math.floor(2.0 * T / E)
    capacity += capacity % 2
    capacity = max(capacity, 4)

    TB = 128
    grid = (T // TB,)
    cw, mask = pl.pallas_call(
        functools.partial(_router_body, capacity),
        grid=grid,
        in_specs=[pl.BlockSpec((T, E), lambda i: (0, 0))],
        out_specs=[
            pl.BlockSpec((TB, E, capacity), lambda i: (i, 0, 0)),
            pl.BlockSpec((TB, E, capacity), lambda i: (i, 0, 0)),
        ],
        out_shape=[
            jax.ShapeDtypeStruct((T, E, capacity), jnp.float32),
            jax.ShapeDtypeStruct((T, E, capacity), jnp.bool_),
        ],
        scratch_shapes=[
            pltpu.VMEM((T, 1), jnp.int32),
            pltpu.VMEM((T, 1), jnp.float32),
            pltpu.VMEM((T, 1), jnp.int32),
            pltpu.VMEM((T, 1), jnp.float32),
        ],
    )(inputs.astype(jnp.float32))
    return cw, mask


# speedup vs baseline: 3.3622x; 3.3622x over previous
import functools
import math

import jax
import jax.numpy as jnp
from jax.experimental import pallas as pl
from jax.experimental.pallas import tpu as pltpu


def _router_body(capacity, xt_ref, cw_ref, mask_ref, w_ref, r_ref):
    i = pl.program_id(0)
    E, T = xt_ref.shape
    EB = cw_ref.shape[0]

    @pl.when(i == 0)
    def _compute_metadata():
        x = xt_ref[...]
        m = jnp.max(x, axis=0, keepdims=True)
        ex = jnp.exp(x - m)
        p = ex / jnp.sum(ex, axis=0, keepdims=True)

        eids = jax.lax.broadcasted_iota(jnp.int32, (E, T), 0)
        p1 = jnp.max(p, axis=0, keepdims=True)
        idx1 = jnp.min(jnp.where(p == p1, eids, E), axis=0, keepdims=True)
        m1 = eids == idx1
        pm = jnp.where(m1, -jnp.inf, p)
        p2 = jnp.max(pm, axis=0, keepdims=True)
        idx2 = jnp.min(jnp.where(pm == p2, eids, E), axis=0, keepdims=True)
        m2 = eids == idx2

        def _cumsum1(v):
            s = 1
            while s < v.shape[1]:
                z = jnp.zeros((v.shape[0], s), v.dtype)
                v = v + jnp.concatenate([z, v[:, :-s]], axis=1)
                s *= 2
            return v

        c1 = _cumsum1(m1.astype(jnp.int32))
        c2 = _cumsum1(m2.astype(jnp.int32))
        total1 = c1[:, T - 1:T]
        rank1 = jnp.sum(jnp.where(m1, c1, 0), axis=0, keepdims=True) - 1
        rank2 = jnp.sum(jnp.where(m2, c2 + total1, 0), axis=0, keepdims=True) - 1

        keep1 = m1 & (rank1 < capacity)
        keep2 = m2 & (rank2 < capacity)
        w_ref[...] = jnp.where(keep1 | keep2, p, 0.0)
        r_ref[...] = (jnp.where(keep1, rank1, -1)
                      + jnp.where(keep2, rank2 + 1, 0))

    w = w_ref[pl.ds(i * EB, EB), :][:, None, :]
    r = r_ref[pl.ds(i * EB, EB), :][:, None, :]
    jc = jax.lax.broadcasted_iota(jnp.int32, (EB, capacity, T), 1)
    out = jnp.where(jc == r, w, 0.0)
    cw_ref[...] = out
    mask_ref[...] = out != 0.0


@jax.jit
def kernel(inputs):
    T, E = inputs.shape
    capacity = math.floor(2.0 * T / E)
    capacity += capacity % 2
    capacity = max(capacity, 4)

    xt = jnp.swapaxes(inputs.astype(jnp.float32), 0, 1)
    EB = 8
    cw_ect, mask_ect = pl.pallas_call(
        functools.partial(_router_body, capacity),
        grid=(E // EB,),
        in_specs=[pl.BlockSpec((E, T), lambda i: (0, 0))],
        out_specs=[
            pl.BlockSpec((EB, capacity, T), lambda i: (i, 0, 0)),
            pl.BlockSpec((EB, capacity, T), lambda i: (i, 0, 0)),
        ],
        out_shape=[
            jax.ShapeDtypeStruct((E, capacity, T), jnp.float32),
            jax.ShapeDtypeStruct((E, capacity, T), jnp.bool_),
        ],
        scratch_shapes=[
            pltpu.VMEM((E, T), jnp.float32),
            pltpu.VMEM((E, T), jnp.int32),
        ],
    )(xt)
    combine_weight = jnp.transpose(cw_ect, (2, 0, 1))
    sec_mask = jnp.transpose(mask_ect, (2, 0, 1))
    return combine_weight, sec_mask
